# baseline (speedup 1.0000x reference)
import functools

import jax
import jax.numpy as jnp
from jax import lax
from jax.experimental import pallas as pl
from jax.experimental.pallas import tpu as pltpu
from jax.experimental.pallas import tpu_sc as plsc

L = 16
NC = 2
NS = 16
NW = NC * NS
CHUNK = 128
NB = 2048
Z0 = -8.0
SCALE = NB / 16.0
INV_SCALE = 16.0 / NB


def _sc_body(F, T, x_hbm, thr_hbm, idx_hbm, out_hbm,
             idx_v, thr_v, table,
             xin0, xin1, yout0, yout1,
             sem_in0, sem_in1, sem_out0, sem_out1):
    rows_per_w = out_hbm.shape[0] // T // NW
    n_chunks = rows_per_w // CHUNK
    ngroups = T // L

    wid = lax.axis_index("s") * NC + lax.axis_index("c")
    base = wid * rows_per_w

    xins = (xin0, xin1)
    youts = (yout0, yout1)
    sin = (sem_in0, sem_in1)
    sout = (sem_out0, sem_out1)

    def in_slice(c):
        return x_hbm.at[pl.ds((base + c * CHUNK) * F, CHUNK * F)]

    def out_slice(c):
        return out_hbm.at[pl.ds((base + c * CHUNK) * T, CHUNK * T)]

    pltpu.async_copy(in_slice(0), xin0, sem_in0)
    pltpu.async_copy(in_slice(1), xin1, sem_in1)

    pltpu.sync_copy(idx_hbm, idx_v)
    pltpu.sync_copy(thr_hbm.at[0], thr_v)

    lane = lax.iota(jnp.int32, 16).astype(jnp.float32)

    @plsc.parallel_loop(0, NB // L, unroll=4)
    def _tb(i):
        zc = Z0 + (lane + (i * 16).astype(jnp.float32) + 0.5) * INV_SCALE
        table[pl.ds(i * L, L)] = 1.0 / (1.0 + jnp.exp(-zc))

    cols = [idx_v[pl.ds(g * L, L)] for g in range(ngroups)]
    offs = [NB / 2.0 - thr_v[pl.ds(g * L, L)] * SCALE for g in range(ngroups)]

    def compute_chunk(xin_b, yout_b):
        @plsc.parallel_loop(0, CHUNK, unroll=8)
        def _row(r):
            rb = r * F
            ro = r * T
            for g in range(ngroups):
                v = plsc.load_gather(xin_b, [cols[g] + rb])
                t = v * SCALE + offs[g]
                t = jnp.minimum(jnp.maximum(t, 0.0), NB - 1.0)
                yout_b[pl.ds(ro + g * L, L)] = plsc.load_gather(
                    table, [t.astype(jnp.int32)])

    def process(c, b, wait_out):
        pltpu.make_async_copy(in_slice(c), xins[b], sin[b]).wait()
        if wait_out:
            pltpu.make_async_copy(youts[b], out_slice(c), sout[b]).wait()
        compute_chunk(xins[b], youts[b])
        pltpu.async_copy(youts[b], out_slice(c), sout[b])

        @pl.when(c + 2 < n_chunks)
        def _():
            pltpu.async_copy(in_slice(c + 2), xins[b], sin[b])

    process(0, 0, False)
    process(1, 1, False)

    def pair(j, _):
        process(2 * j, 0, True)
        process(2 * j + 1, 1, True)
        return 0

    lax.fori_loop(1, n_chunks // 2, pair, 0)

    pltpu.make_async_copy(youts[0], out_slice(n_chunks - 2), sout[0]).wait()
    pltpu.make_async_copy(youts[1], out_slice(n_chunks - 1), sout[1]).wait()


def kernel(x, thresholds, thresholds_index):
    B, F = x.shape
    T = thresholds.shape[1]
    mesh = plsc.VectorSubcoreMesh(
        core_axis_name="c", subcore_axis_name="s", num_cores=NC, num_subcores=NS
    )
    run = pl.kernel(
        functools.partial(_sc_body, F, T),
        out_type=jax.ShapeDtypeStruct((B * T,), jnp.float32),
        mesh=mesh,
        scratch_types=[
            pltpu.VMEM((T,), jnp.int32),
            pltpu.VMEM((T,), jnp.float32),
            pltpu.VMEM((NB,), jnp.float32),
            pltpu.VMEM((CHUNK * F,), jnp.float32),
            pltpu.VMEM((CHUNK * F,), jnp.float32),
            pltpu.VMEM((CHUNK * T,), jnp.float32),
            pltpu.VMEM((CHUNK * T,), jnp.float32),
            pltpu.SemaphoreType.DMA,
            pltpu.SemaphoreType.DMA,
            pltpu.SemaphoreType.DMA,
            pltpu.SemaphoreType.DMA,
        ],
        compiler_params=pltpu.CompilerParams(needs_layout_passes=False),
    )
    return run(x.reshape(-1), thresholds, thresholds_index).reshape(B, T)


# speedup vs baseline: 1.3770x; 1.3770x over previous
import functools

import jax
import jax.numpy as jnp
from jax import lax
from jax.experimental import pallas as pl
from jax.experimental.pallas import tpu as pltpu
from jax.experimental.pallas import tpu_sc as plsc

L = 16
NC = 2
NS = 16
NW = NC * NS
CHUNK = 128
NB = 2048
Z0 = -8.0
SCALE = NB / 16.0
INV_SCALE = 16.0 / NB


def _sc_body(F, T, x_hbm, thr_hbm, idx_hbm, out_hbm,
             idx_v, thr_v, table,
             xin0, xin1, yout0, yout1,
             sem_in0, sem_in1, sem_out0, sem_out1):
    rows_per_w = out_hbm.shape[0] // T // NW
    n_chunks = rows_per_w // CHUNK
    ngroups = T // L

    wid = lax.axis_index("s") * NC + lax.axis_index("c")
    base = wid * rows_per_w

    xins = (xin0, xin1)
    youts = (yout0, yout1)
    sin = (sem_in0, sem_in1)
    sout = (sem_out0, sem_out1)

    def in_slice(c):
        return x_hbm.at[pl.ds((base + c * CHUNK) * F, CHUNK * F)]

    def out_slice(c):
        return out_hbm.at[pl.ds((base + c * CHUNK) * T, CHUNK * T)]

    pltpu.async_copy(in_slice(0), xin0, sem_in0)
    pltpu.async_copy(in_slice(1), xin1, sem_in1)

    pltpu.sync_copy(idx_hbm, idx_v)
    pltpu.sync_copy(thr_hbm.at[0], thr_v)

    lane = lax.iota(jnp.int32, 16).astype(jnp.float32)

    @plsc.parallel_loop(0, NB // L, unroll=4)
    def _tb(i):
        zc = Z0 + (lane + (i * 16).astype(jnp.float32) + 0.5) * INV_SCALE
        table[pl.ds(i * L, L)] = 1.0 / (1.0 + jnp.exp(-zc))

    def compute_chunk(xin_b, yout_b):
        for g in range(ngroups):
            cols_g = idx_v[pl.ds(g * L, L)]
            offs_g = NB / 2.0 - thr_v[pl.ds(g * L, L)] * SCALE

            @plsc.parallel_loop(0, CHUNK, unroll=4)
            def _row(r, cols_g=cols_g, offs_g=offs_g, g=g):
                v = plsc.load_gather(xin_b.at[pl.ds(r * F, F)], [cols_g])
                t = v * SCALE + offs_g
                t = jnp.minimum(jnp.maximum(t, 0.0), NB - 1.0)
                yout_b[pl.ds(r * T + g * L, L)] = plsc.load_gather(
                    table, [t.astype(jnp.int32)])

    def process(c, b, wait_out):
        pltpu.make_async_copy(in_slice(c), xins[b], sin[b]).wait()
        if wait_out:
            pltpu.make_async_copy(youts[b], out_slice(c), sout[b]).wait()
        compute_chunk(xins[b], youts[b])
        pltpu.async_copy(youts[b], out_slice(c), sout[b])

        @pl.when(c + 2 < n_chunks)
        def _():
            pltpu.async_copy(in_slice(c + 2), xins[b], sin[b])

    process(0, 0, False)
    process(1, 1, False)

    def pair(j, _):
        process(2 * j, 0, True)
        process(2 * j + 1, 1, True)
        return 0

    lax.fori_loop(1, n_chunks // 2, pair, 0)

    pltpu.make_async_copy(youts[0], out_slice(n_chunks - 2), sout[0]).wait()
    pltpu.make_async_copy(youts[1], out_slice(n_chunks - 1), sout[1]).wait()


def kernel(x, thresholds, thresholds_index):
    B, F = x.shape
    T = thresholds.shape[1]
    mesh = plsc.VectorSubcoreMesh(
        core_axis_name="c", subcore_axis_name="s", num_cores=NC, num_subcores=NS
    )
    run = pl.kernel(
        functools.partial(_sc_body, F, T),
        out_type=jax.ShapeDtypeStruct((B * T,), jnp.float32),
        mesh=mesh,
        scratch_types=[
            pltpu.VMEM((T,), jnp.int32),
            pltpu.VMEM((T,), jnp.float32),
            pltpu.VMEM((NB,), jnp.float32),
            pltpu.VMEM((CHUNK * F,), jnp.float32),
            pltpu.VMEM((CHUNK * F,), jnp.float32),
            pltpu.VMEM((CHUNK * T,), jnp.float32),
            pltpu.VMEM((CHUNK * T,), jnp.float32),
            pltpu.SemaphoreType.DMA,
            pltpu.SemaphoreType.DMA,
            pltpu.SemaphoreType.DMA,
            pltpu.SemaphoreType.DMA,
        ],
        compiler_params=pltpu.CompilerParams(needs_layout_passes=False),
    )
    return run(x.reshape(-1), thresholds, thresholds_index).reshape(B, T)


# speedup vs baseline: 1.3772x; 1.0002x over previous
import functools

import jax
import jax.numpy as jnp
from jax import lax
from jax.experimental import pallas as pl
from jax.experimental.pallas import tpu as pltpu
from jax.experimental.pallas import tpu_sc as plsc

L = 16
NC = 2
NS = 16
NW = NC * NS
CHUNK = 128
NB = 2048
Z0 = -8.0
SCALE = NB / 16.0
INV_SCALE = 16.0 / NB


def _sc_body(F, T, x_hbm, thr_hbm, idx_hbm, out_hbm,
             idx_v, thr_v, table,
             xin0, xin1, yout0, yout1,
             sem_in0, sem_in1, sem_out0, sem_out1):
    rows_per_w = out_hbm.shape[0] // T // NW
    n_chunks = rows_per_w // CHUNK
    ngroups = T // L

    wid = lax.axis_index("s") * NC + lax.axis_index("c")
    base = wid * rows_per_w

    xins = (xin0, xin1)
    youts = (yout0, yout1)
    sin = (sem_in0, sem_in1)
    sout = (sem_out0, sem_out1)

    def in_slice(c):
        return x_hbm.at[pl.ds((base + c * CHUNK) * F, CHUNK * F)]

    def out_slice(c):
        return out_hbm.at[pl.ds((base + c * CHUNK) * T, CHUNK * T)]

    pltpu.async_copy(in_slice(0), xin0, sem_in0)
    pltpu.async_copy(in_slice(1), xin1, sem_in1)

    pltpu.sync_copy(idx_hbm, idx_v)
    pltpu.sync_copy(thr_hbm.at[0], thr_v)

    lane = lax.iota(jnp.int32, 16).astype(jnp.float32)

    @plsc.parallel_loop(0, NB // L, unroll=4)
    def _tb(i):
        zc = Z0 + (lane + (i * 16).astype(jnp.float32) + 0.5) * INV_SCALE
        table[pl.ds(i * L, L)] = 1.0 / (1.0 + jnp.exp(-zc))

    def compute_chunk(xin_b, yout_b):
        for g in range(ngroups):
            cols_g = idx_v[pl.ds(g * L, L)]
            offs_g = NB / 2.0 - thr_v[pl.ds(g * L, L)] * SCALE

            @plsc.parallel_loop(0, CHUNK, unroll=4)
            def _row(r, cols_g=cols_g, offs_g=offs_g, g=g):
                v = plsc.load_gather(xin_b.at[pl.ds(r * F, F)], [cols_g])
                t = v * SCALE + offs_g
                t = jnp.minimum(jnp.maximum(t, 0.0), NB - 1.0)
                yout_b[pl.ds(r * T + g * L, L)] = plsc.load_gather(
                    table, [t.astype(jnp.int32)])

    def process(c, b, wait_out):
        pltpu.make_async_copy(in_slice(c), xins[b], sin[b]).wait()
        if wait_out:
            pltpu.make_async_copy(youts[b], out_slice(c), sout[b]).wait()
        compute_chunk(xins[b], youts[b])
        pltpu.async_copy(youts[b], out_slice(c), sout[b])

        @pl.when(c + 2 < n_chunks)
        def _():
            pltpu.async_copy(in_slice(c + 2), xins[b], sin[b])

    process(0, 0, False)
    process(1, 1, False)

    def pair(j, _):
        process(2 * j, 0, True)
        process(2 * j + 1, 1, True)
        return 0

    lax.fori_loop(1, n_chunks // 2, pair, 0)

    pltpu.make_async_copy(youts[0], out_slice(n_chunks - 2), sout[0]).wait()
    pltpu.make_async_copy(youts[1], out_slice(n_chunks - 1), sout[1]).wait()


def kernel(x, thresholds, thresholds_index):
    B, F = x.shape
    T = thresholds.shape[1]
    mesh = plsc.VectorSubcoreMesh(
        core_axis_name="c", subcore_axis_name="s", num_cores=NC, num_subcores=NS
    )
    run = pl.kernel(
        functools.partial(_sc_body, F, T),
        out_type=jax.ShapeDtypeStruct((B * T,), jnp.float32),
        mesh=mesh,
        scratch_types=[
            pltpu.VMEM((T,), jnp.int32),
            pltpu.VMEM((T,), jnp.float32),
            pltpu.VMEM((NB,), jnp.float32),
            pltpu.VMEM((CHUNK * F,), jnp.float32),
            pltpu.VMEM((CHUNK * F,), jnp.float32),
            pltpu.VMEM((CHUNK * T,), jnp.float32),
            pltpu.VMEM((CHUNK * T,), jnp.float32),
            pltpu.SemaphoreType.DMA,
            pltpu.SemaphoreType.DMA,
            pltpu.SemaphoreType.DMA,
            pltpu.SemaphoreType.DMA,
        ],
        compiler_params=pltpu.CompilerParams(
            needs_layout_passes=False,
            disable_bounds_checks=True,
            disable_semaphore_checks=True,
        ),
    )
    return run(x.reshape(-1), thresholds, thresholds_index).reshape(B, T)
